# baseline (speedup 1.0000x reference)
import functools

import jax
import jax.numpy as jnp
from jax import lax
from jax.experimental import pallas as pl
from jax.experimental.pallas import tpu as pltpu
from jax.experimental.pallas import tpu_sc as plsc

_N = 100000
_NC = 2
_NS = 16
_NW = _NC * _NS
_LANES = 16
_FULL = 3136
_LAST = _N - (_NW - 1) * _FULL
_T = 16


@functools.cache
def _build():
    @functools.partial(
        pl.kernel,
        mesh=plsc.VectorSubcoreMesh(core_axis_name="c", subcore_axis_name="s"),
        out_type=jax.ShapeDtypeStruct((_N,), jnp.float32),
        scratch_types=[
            pltpu.VMEM((_FULL,), jnp.float32),
            pltpu.VMEM((_FULL,), jnp.int32),
            pltpu.VMEM((_FULL,), jnp.float32),
            pltpu.VMEM((_T,), jnp.float32),
            pltpu.VMEM((_T,), jnp.float32),
            pltpu.SemaphoreType.DMA,
            pltpu.SemaphoreType.DMA,
            pltpu.SemaphoreType.DMA,
        ],
    )
    def _shift_scale(x_hbm, t_hbm, scale_hbm, shift_hbm, out_hbm,
                     x_v, t_v, o_v, scale_v, shift_v, sem_a, sem_b, sem_o):
        wid = lax.axis_index("s") * _NC + lax.axis_index("c")
        base = wid * _FULL

        def compute(v_off, n_elems, scale_vec, shift_vec):
            nv = n_elems // _LANES
            nv_main = (nv // 4) * 4

            def step(i):
                sl = pl.ds(v_off + i * _LANES, _LANES)
                t = t_v[sl]
                s = scale_vec.at[t].get(mode="promise_in_bounds")
                h = shift_vec.at[t].get(mode="promise_in_bounds")
                o_v[sl] = s * x_v[sl] + h

            plsc.parallel_loop(0, nv_main, unroll=4)(step)
            for i in range(nv_main, nv):
                step(i)

        def do_chunk(n_elems):
            na = (n_elems // 2) & ~127
            nb = n_elems - na
            sl_ha = pl.ds(base, na)
            sl_hb = pl.ds(base + na, nb)
            sl_va = pl.ds(0, na)
            sl_vb = pl.ds(na, nb)
            a1 = pltpu.async_copy(x_hbm.at[sl_ha], x_v.at[sl_va], sem_a)
            a2 = pltpu.async_copy(t_hbm.at[sl_ha], t_v.at[sl_va], sem_a)
            a3 = pltpu.async_copy(scale_hbm, scale_v, sem_a)
            a4 = pltpu.async_copy(shift_hbm, shift_v, sem_a)
            b1 = pltpu.async_copy(x_hbm.at[sl_hb], x_v.at[sl_vb], sem_b)
            b2 = pltpu.async_copy(t_hbm.at[sl_hb], t_v.at[sl_vb], sem_b)
            a1.wait()
            a2.wait()
            a3.wait()
            a4.wait()
            scale_vec = scale_v[...]
            shift_vec = shift_v[...]
            compute(0, na, scale_vec, shift_vec)
            oa = pltpu.async_copy(o_v.at[sl_va], out_hbm.at[sl_ha], sem_o)
            b1.wait()
            b2.wait()
            compute(na, nb, scale_vec, shift_vec)
            ob = pltpu.async_copy(o_v.at[sl_vb], out_hbm.at[sl_hb], sem_o)
            oa.wait()
            ob.wait()

        @pl.when(wid < _NW - 1)
        def _():
            do_chunk(_FULL)

        @pl.when(wid == _NW - 1)
        def _():
            do_chunk(_LAST)

    return _shift_scale


def kernel(x, atom_type, scale, shift):
    return _build()(x, atom_type.astype(jnp.int32), scale, shift)
